# baseline (speedup 1.0000x reference)
import functools

import jax
import jax.numpy as jnp
from jax import lax
from jax.experimental import pallas as pl
from jax.experimental.pallas import tpu as pltpu
from jax.experimental.pallas import tpu_sc as plsc

_CHUNK = 32
_NBUF = 3


@functools.lru_cache(maxsize=None)
def _make_gather(V, D, B):
  info = plsc.get_sparse_core_info()
  nw = info.num_cores * info.num_subcores
  assert B % nw == 0
  b_per_w = B // nw
  assert b_per_w % _CHUNK == 0
  nchunks = b_per_w // _CHUNK
  mesh = plsc.VectorSubcoreMesh(core_axis_name="c", subcore_axis_name="s")

  @functools.partial(
      pl.kernel,
      mesh=mesh,
      out_type=jax.ShapeDtypeStruct((B, D), jnp.float32),
      scratch_types=(
          [pltpu.VMEM((b_per_w,), jnp.int32)]
          + [pltpu.VMEM((_CHUNK, D), jnp.float32) for _ in range(_NBUF)]
          + [pltpu.SemaphoreType.DMA for _ in range(2 * _NBUF)]
      ),
  )
  def gather_kernel(table_hbm, idx_hbm, out_hbm, idx_v, *rest):
    bufs = rest[:_NBUF]
    gsems = rest[_NBUF:2 * _NBUF]
    ssems = rest[2 * _NBUF:]
    wid = lax.axis_index("s") * info.num_cores + lax.axis_index("c")
    base = wid * b_per_w
    pltpu.sync_copy(idx_hbm.at[pl.ds(base, b_per_w)], idx_v)

    def start_gather(g):
      s = g % _NBUF
      pltpu.async_copy(
          table_hbm.at[idx_v.at[pl.ds(g * _CHUNK, _CHUNK)]], bufs[s], gsems[s])

    def wait_gather(g):
      s = g % _NBUF
      pltpu.make_async_copy(
          table_hbm.at[idx_v.at[pl.ds(g * _CHUNK, _CHUNK)]], bufs[s],
          gsems[s]).wait()

    def start_store(g):
      s = g % _NBUF
      pltpu.async_copy(
          bufs[s], out_hbm.at[pl.ds(base + g * _CHUNK, _CHUNK)], ssems[s])

    def wait_store(g):
      s = g % _NBUF
      pltpu.make_async_copy(
          bufs[s], out_hbm.at[pl.ds(base + g * _CHUNK, _CHUNK)],
          ssems[s]).wait()

    for g in range(min(_NBUF, nchunks)):
      start_gather(g)
    for g in range(nchunks):
      wait_gather(g)
      start_store(g)
      pg = g - 1
      ng = pg + _NBUF
      if pg >= 0 and ng < nchunks:
        wait_store(pg)
        start_gather(ng)
    for g in range(max(0, nchunks - _NBUF), nchunks):
      wait_store(g)

  return gather_kernel


def kernel(pe, index):
  V, D = pe.shape
  (B,) = index.shape
  return _make_gather(V, D, B)(pe, index.astype(jnp.int32))


# speedup vs baseline: 1.0189x; 1.0189x over previous
import functools

import jax
import jax.numpy as jnp
from jax import lax
from jax.experimental import pallas as pl
from jax.experimental.pallas import tpu as pltpu
from jax.experimental.pallas import tpu_sc as plsc

_CHUNK = 16
_NBUF = 6


@functools.lru_cache(maxsize=None)
def _make_gather(V, D, B):
  info = plsc.get_sparse_core_info()
  nw = info.num_cores * info.num_subcores
  assert B % nw == 0
  b_per_w = B // nw
  assert b_per_w % _CHUNK == 0
  nchunks = b_per_w // _CHUNK
  mesh = plsc.VectorSubcoreMesh(core_axis_name="c", subcore_axis_name="s")

  @functools.partial(
      pl.kernel,
      mesh=mesh,
      out_type=jax.ShapeDtypeStruct((B, D), jnp.float32),
      scratch_types=(
          [pltpu.VMEM((b_per_w,), jnp.int32)]
          + [pltpu.VMEM((_CHUNK, D), jnp.float32) for _ in range(_NBUF)]
          + [pltpu.SemaphoreType.DMA for _ in range(2 * _NBUF)]
      ),
  )
  def gather_kernel(table_hbm, idx_hbm, out_hbm, idx_v, *rest):
    bufs = rest[:_NBUF]
    gsems = rest[_NBUF:2 * _NBUF]
    ssems = rest[2 * _NBUF:]
    wid = lax.axis_index("s") * info.num_cores + lax.axis_index("c")
    base = wid * b_per_w
    pltpu.sync_copy(idx_hbm.at[pl.ds(base, b_per_w)], idx_v)

    def start_gather(g):
      s = g % _NBUF
      pltpu.async_copy(
          table_hbm.at[idx_v.at[pl.ds(g * _CHUNK, _CHUNK)]], bufs[s], gsems[s])

    def wait_gather(g):
      s = g % _NBUF
      pltpu.make_async_copy(
          table_hbm.at[idx_v.at[pl.ds(g * _CHUNK, _CHUNK)]], bufs[s],
          gsems[s]).wait()

    def start_store(g):
      s = g % _NBUF
      pltpu.async_copy(
          bufs[s], out_hbm.at[pl.ds(base + g * _CHUNK, _CHUNK)], ssems[s])

    def wait_store(g):
      s = g % _NBUF
      pltpu.make_async_copy(
          bufs[s], out_hbm.at[pl.ds(base + g * _CHUNK, _CHUNK)],
          ssems[s]).wait()

    for g in range(min(_NBUF, nchunks)):
      start_gather(g)
    for g in range(nchunks):
      wait_gather(g)
      start_store(g)
      pg = g - 1
      ng = pg + _NBUF
      if pg >= 0 and ng < nchunks:
        wait_store(pg)
        start_gather(ng)
    for g in range(max(0, nchunks - _NBUF), nchunks):
      wait_store(g)

  return gather_kernel


def kernel(pe, index):
  V, D = pe.shape
  (B,) = index.shape
  return _make_gather(V, D, B)(pe, index.astype(jnp.int32))
